# baseline (speedup 1.0000x reference)
import functools

import jax
import jax.numpy as jnp
from jax import lax
from jax.experimental import pallas as pl
from jax.experimental.pallas import tpu as pltpu
from jax.experimental.pallas import tpu_sc as plsc

N = 10000
E = 320000
NEG_SLOPE = 0.01

ROW_BLK = 2000

NC = 2
NS = 16
CHUNK = 128
NCH_C = 162
NCH_E = 81
E_PAD = NS * NCH_C * CHUNK
ROWS_PER_SUB = 632
N_ACC = NS * ROWS_PER_SUB
DST_PAD = N + 1


def _leaky(y):
    return jnp.where(y >= 0, y, NEG_SLOPE * y)


def _tc_layer_body(nh, na, agg_mode, h_and_agg_and_w, outs):
    refs = h_and_agg_and_w
    i = pl.program_id(0)
    h_parts = [refs[k][...] for k in range(nh)]
    agg_parts = [refs[nh + k][...] for k in range(na)]
    w_ref = refs[nh + na]
    b_ref = refs[nh + na + 1]
    eps_ref = refs[nh + na + 2]
    h = h_parts[0] if nh == 1 else jnp.concatenate(h_parts, axis=1)
    if na == 1:
        agg = agg_parts[0]
    elif agg_mode == "sum":
        agg = agg_parts[0] + agg_parts[1]
    else:
        agg = jnp.concatenate(agg_parts, axis=1)
    rst = eps_ref[0, 0] * h + agg
    y = _leaky(jnp.dot(rst, w_ref[...], preferred_element_type=jnp.float32)
               + b_ref[...])
    y0_ref, y1_ref, s_ref, m_ref = outs
    H2 = y.shape[1] // 2
    y0_ref[...] = y[:, :H2]
    y1_ref[...] = y[:, H2:]
    bsum = jnp.sum(y, axis=0, keepdims=True)
    bmax = jnp.max(y, axis=0, keepdims=True)

    @pl.when(i == 0)
    def _init():
        s_ref[...] = bsum
        m_ref[...] = bmax

    @pl.when(i > 0)
    def _acc():
        s_ref[...] += bsum
        m_ref[...] = jnp.maximum(m_ref[...], bmax)


def _tc_layer(h_parts, agg_parts, W, b, eps1, agg_mode="concat"):
    nh, na = len(h_parts), len(agg_parts)
    D = sum(p.shape[1] for p in h_parts)
    H = W.shape[1]
    grid = (N // ROW_BLK,)
    in_specs = (
        [pl.BlockSpec((ROW_BLK, p.shape[1]), lambda i: (i, 0)) for p in h_parts]
        + [pl.BlockSpec((ROW_BLK, p.shape[1]), lambda i: (i, 0)) for p in agg_parts]
        + [pl.BlockSpec((D, H), lambda i: (0, 0)),
           pl.BlockSpec((1, H), lambda i: (0, 0)),
           pl.BlockSpec((1, 1), lambda i: (0, 0), memory_space=pltpu.SMEM)]
    )
    out_specs = [
        pl.BlockSpec((ROW_BLK, H // 2), lambda i: (i, 0)),
        pl.BlockSpec((ROW_BLK, H // 2), lambda i: (i, 0)),
        pl.BlockSpec((1, H), lambda i: (0, 0)),
        pl.BlockSpec((1, H), lambda i: (0, 0)),
    ]
    out_shape = [
        jax.ShapeDtypeStruct((N, H // 2), jnp.float32),
        jax.ShapeDtypeStruct((N, H // 2), jnp.float32),
        jax.ShapeDtypeStruct((1, H), jnp.float32),
        jax.ShapeDtypeStruct((1, H), jnp.float32),
    ]

    def body(*refs):
        _tc_layer_body(nh, na, agg_mode, refs[:-4], refs[-4:])

    return pl.pallas_call(
        body,
        grid=grid,
        in_specs=in_specs,
        out_specs=out_specs,
        out_shape=out_shape,
    )(*h_parts, *agg_parts, W, b, eps1)


def _head_body(m_ref, w1_ref, b1_ref, w2_ref, b2_ref, o_ref):
    d1 = _leaky(jnp.dot(m_ref[...], w1_ref[...],
                        preferred_element_type=jnp.float32) + b1_ref[...])
    d2 = jnp.dot(d1, w2_ref[...], preferred_element_type=jnp.float32) + b2_ref[...]
    o_ref[...] = jax.nn.sigmoid(d2)


def _head(merged, Wd1, bd1, Wd2, bd2):
    return pl.pallas_call(
        _head_body,
        out_shape=jax.ShapeDtypeStruct((1, 2), jnp.float32),
    )(merged, Wd1, bd1[None, :], Wd2, bd2[None, :])


@functools.cache
def _make_sc_segsum(nrows, nch):
    mesh = plsc.VectorSubcoreMesh(core_axis_name="c", subcore_axis_name="s")

    @functools.partial(
        pl.kernel,
        out_type=jax.ShapeDtypeStruct((NC * N_ACC, 128), jnp.float32),
        mesh=mesh,
        scratch_types=[
            pltpu.VMEM((CHUNK,), jnp.int32),
            pltpu.VMEM((CHUNK,), jnp.int32),
            pltpu.VMEM((CHUNK,), jnp.int32),
            pltpu.VMEM((CHUNK,), jnp.int32),
            pltpu.VMEM((CHUNK,), jnp.int32),
            pltpu.VMEM((CHUNK,), jnp.int32),
            pltpu.VMEM((CHUNK, 128), jnp.float32),
            pltpu.VMEM((CHUNK, 128), jnp.float32),
            pltpu.VMEM((CHUNK, 128), jnp.float32),
            pltpu.VMEM_SHARED((N_ACC, 128), jnp.float32),
            pltpu.SemaphoreType.DMA,
            pltpu.SemaphoreType.DMA,
            pltpu.SemaphoreType.DMA,
            pltpu.SemaphoreType.DMA,
            pltpu.SemaphoreType.DMA,
            pltpu.SemaphoreType.DMA,
        ],
    )
    def segsum(src4, dst4, hstack, out, srcv0, srcv1, srcv2,
               dstv0, dstv1, dstv2, rows0, rows1, rows2, acc,
               semi0, semi1, semi2, semg0, semg1, semg2):
        c = lax.axis_index("c")
        s = lax.axis_index("s")
        srcv = (srcv0, srcv1, srcv2)
        dstv = (dstv0, dstv1, dstv2)
        rows = (rows0, rows1, rows2)
        semi = (semi0, semi1, semi2)
        semg = (semg0, semg1, semg2)
        zeros16 = jnp.zeros((16,), jnp.float32)

        def _idx_copy(jj, b):
            return (pltpu.make_async_copy(src4.at[c, s, jj], srcv[b], semi[b]),
                    pltpu.make_async_copy(dst4.at[c, s, jj], dstv[b], semi[b]))

        def _gather(b):
            return pltpu.make_async_copy(
                hstack.at[srcv[b]], rows[b], semg[b])

        def _zero_row(r, carry):
            for k0 in range(8):
                rows0[r, pl.ds(k0 * 16, 16)] = zeros16
            return carry

        lax.fori_loop(0, CHUNK, _zero_row, 0)

        base = s * ROWS_PER_SUB
        pieces = []
        off = 0
        while off < ROWS_PER_SUB:
            sz = min(CHUNK, ROWS_PER_SUB - off)
            pieces.append((off, sz))
            off += sz
        for off, sz in pieces:
            pltpu.sync_copy(rows0.at[pl.ds(0, sz)], acc.at[pl.ds(base + off, sz)])
        plsc.subcore_barrier()

        for bb in range(3):
            for cp in _idx_copy(bb, bb):
                cp.start()
        for cp in _idx_copy(0, 0):
            cp.wait()
        _gather(0).start()
        for cp in _idx_copy(1, 1):
            cp.wait()
        _gather(1).start()

        def _edge_trip(i, carry):
            for b in range(3):
                jj = 3 * i + b
                nb2 = (b + 2) % 3

                @pl.when(jj + 2 < nch)
                def _next_gather():
                    for cp in _idx_copy(jj + 2, nb2):
                        cp.wait()
                    _gather(nb2).start()

                _gather(b).wait()
                pltpu.sync_copy(rows[b], acc.at[dstv[b]], add=True)

                @pl.when(jj + 3 < nch)
                def _next_idx():
                    for cp in _idx_copy(jj + 3, b):
                        cp.start()
            return carry

        lax.fori_loop(0, nch // 3, _edge_trip, 0)
        plsc.subcore_barrier()

        obase = c * N_ACC + base
        for off, sz in pieces:
            pltpu.sync_copy(acc.at[pl.ds(base + off, sz)], rows0.at[pl.ds(0, sz)])
            pltpu.sync_copy(rows0.at[pl.ds(0, sz)], out.at[pl.ds(obase + off, sz)])

    return segsum


def _edge_prep(edge_index):
    src = edge_index[0]
    dst = edge_index[1]
    srcp = jnp.concatenate([src, jnp.zeros((E_PAD - E,), jnp.int32)])
    pad_dst = N + jnp.arange(E_PAD - E, dtype=jnp.int32) % (N_ACC - N)
    dstp = jnp.concatenate([dst, pad_dst])
    src_r = srcp.reshape(NS, NCH_C, CHUNK)
    dst_r = dstp.reshape(NS, NCH_C, CHUNK)
    sd_c = (jnp.stack([src_r, src_r + N]), jnp.stack([dst_r, dst_r]))
    sd_e = (srcp.reshape(NC, NS, NCH_E, CHUNK),
            dstp.reshape(NC, NS, NCH_E, CHUNK))
    return sd_c, sd_e


def _segsum_parts(h_parts, sd_c, sd_e):
    if len(h_parts) == 1:
        out = _make_sc_segsum(N, NCH_E)(*sd_e, h_parts[0])
        return [out[:N], out[N_ACC:N_ACC + N]], "sum"
    hstack = jnp.concatenate(h_parts, axis=0)
    out = _make_sc_segsum(2 * N, NCH_C)(*sd_c, hstack)
    return [out[:N], out[N_ACC:N_ACC + N]], "concat"


def kernel(x, edge_index, Wg0, bg0, eps0, Wg1, bg1, eps1, Wg2, bg2, eps2,
           Wd1, bd1, Wd2, bd2):
    edges_c, edges_e = _edge_prep(edge_index)
    readouts = []
    h_parts = [x]
    for (W, b, eps) in ((Wg0, bg0, eps0), (Wg1, bg1, eps1), (Wg2, bg2, eps2)):
        agg_parts, agg_mode = _segsum_parts(h_parts, edges_c, edges_e)
        e1 = jnp.reshape(1.0 + eps, (1, 1))
        y0, y1, s, m = _tc_layer(h_parts, agg_parts, W, b[None, :], e1,
                                 agg_mode=agg_mode)
        readouts.extend([s, m])
        h_parts = [y0, y1]
    merged = jnp.concatenate(readouts, axis=1)
    return _head(merged, Wd1, bd1, Wd2, bd2)


# speedup vs baseline: 3.6781x; 3.6781x over previous
import functools

import jax
import jax.numpy as jnp
from jax import lax
from jax.experimental import pallas as pl
from jax.experimental.pallas import tpu as pltpu
from jax.experimental.pallas import tpu_sc as plsc

N = 10000
E = 320000
NEG_SLOPE = 0.01

ROW_BLK = 2000

NC = 2
NS = 16
CHUNK = 128
NCH_C = 162
NCH_E = 81
E_PAD = NS * NCH_C * CHUNK
ROWS_PER_SUB = 632
N_ACC = NS * ROWS_PER_SUB
DST_PAD = N + 1


def _leaky(y):
    return jnp.where(y >= 0, y, NEG_SLOPE * y)


def _tc_layer_body(nh, na, agg_mode, h_and_agg_and_w, outs):
    refs = h_and_agg_and_w
    i = pl.program_id(0)
    h_parts = [refs[k][...] for k in range(nh)]
    agg_parts = [refs[nh + k][...] for k in range(na)]
    w_ref = refs[nh + na]
    b_ref = refs[nh + na + 1]
    eps_ref = refs[nh + na + 2]
    h = h_parts[0] if nh == 1 else jnp.concatenate(h_parts, axis=1)
    if na == 1:
        agg = agg_parts[0]
    elif agg_mode == "sum":
        agg = agg_parts[0] + agg_parts[1]
    else:
        agg = jnp.concatenate(agg_parts, axis=1)
    rst = eps_ref[0, 0] * h + agg
    y = _leaky(jnp.dot(rst, w_ref[...], preferred_element_type=jnp.float32)
               + b_ref[...])
    y0_ref, y1_ref, s_ref, m_ref = outs
    H2 = y.shape[1] // 2
    y0_ref[...] = y[:, :H2]
    y1_ref[...] = y[:, H2:]
    bsum = jnp.sum(y, axis=0, keepdims=True)
    bmax = jnp.max(y, axis=0, keepdims=True)

    @pl.when(i == 0)
    def _init():
        s_ref[...] = bsum
        m_ref[...] = bmax

    @pl.when(i > 0)
    def _acc():
        s_ref[...] += bsum
        m_ref[...] = jnp.maximum(m_ref[...], bmax)


def _tc_layer(h_parts, agg_parts, W, b, eps1, agg_mode="concat"):
    nh, na = len(h_parts), len(agg_parts)
    D = sum(p.shape[1] for p in h_parts)
    H = W.shape[1]
    grid = (N // ROW_BLK,)
    in_specs = (
        [pl.BlockSpec((ROW_BLK, p.shape[1]), lambda i: (i, 0)) for p in h_parts]
        + [pl.BlockSpec((ROW_BLK, p.shape[1]), lambda i: (i, 0)) for p in agg_parts]
        + [pl.BlockSpec((D, H), lambda i: (0, 0)),
           pl.BlockSpec((1, H), lambda i: (0, 0)),
           pl.BlockSpec((1, 1), lambda i: (0, 0), memory_space=pltpu.SMEM)]
    )
    out_specs = [
        pl.BlockSpec((ROW_BLK, H // 2), lambda i: (i, 0)),
        pl.BlockSpec((ROW_BLK, H // 2), lambda i: (i, 0)),
        pl.BlockSpec((1, H), lambda i: (0, 0)),
        pl.BlockSpec((1, H), lambda i: (0, 0)),
    ]
    out_shape = [
        jax.ShapeDtypeStruct((N, H // 2), jnp.float32),
        jax.ShapeDtypeStruct((N, H // 2), jnp.float32),
        jax.ShapeDtypeStruct((1, H), jnp.float32),
        jax.ShapeDtypeStruct((1, H), jnp.float32),
    ]

    def body(*refs):
        _tc_layer_body(nh, na, agg_mode, refs[:-4], refs[-4:])

    return pl.pallas_call(
        body,
        grid=grid,
        in_specs=in_specs,
        out_specs=out_specs,
        out_shape=out_shape,
    )(*h_parts, *agg_parts, W, b, eps1)


def _head_body(m_ref, w1_ref, b1_ref, w2_ref, b2_ref, o_ref):
    d1 = _leaky(jnp.dot(m_ref[...], w1_ref[...],
                        preferred_element_type=jnp.float32) + b1_ref[...])
    d2 = jnp.dot(d1, w2_ref[...], preferred_element_type=jnp.float32) + b2_ref[...]
    o_ref[...] = jax.nn.sigmoid(d2)


def _head(merged, Wd1, bd1, Wd2, bd2):
    return pl.pallas_call(
        _head_body,
        out_shape=jax.ShapeDtypeStruct((1, 2), jnp.float32),
    )(merged, Wd1, bd1[None, :], Wd2, bd2[None, :])


@functools.cache
def _make_sc_segsum(nrows, nch):
    mesh = plsc.VectorSubcoreMesh(core_axis_name="c", subcore_axis_name="s")

    @functools.partial(
        pl.kernel,
        out_type=jax.ShapeDtypeStruct((NC * N_ACC, 128), jnp.float32),
        mesh=mesh,
        scratch_types=[
            pltpu.VMEM((CHUNK,), jnp.int32),
            pltpu.VMEM((CHUNK,), jnp.int32),
            pltpu.VMEM((CHUNK,), jnp.int32),
            pltpu.VMEM((CHUNK,), jnp.int32),
            pltpu.VMEM((CHUNK,), jnp.int32),
            pltpu.VMEM((CHUNK,), jnp.int32),
            pltpu.VMEM((CHUNK, 128), jnp.float32),
            pltpu.VMEM((CHUNK, 128), jnp.float32),
            pltpu.VMEM((CHUNK, 128), jnp.float32),
            pltpu.VMEM_SHARED((N_ACC, 128), jnp.float32),
            pltpu.SemaphoreType.DMA,
            pltpu.SemaphoreType.DMA,
            pltpu.SemaphoreType.DMA,
            pltpu.SemaphoreType.DMA,
            pltpu.SemaphoreType.DMA,
            pltpu.SemaphoreType.DMA,
        ],
    )
    def segsum(src4, dst4, hstack, out, srcv0, srcv1, srcv2,
               dstv0, dstv1, dstv2, rows0, rows1, rows2, acc,
               semi0, semi1, semi2, semg0, semg1, semg2):
        c = lax.axis_index("c")
        s = lax.axis_index("s")
        srcv = (srcv0, srcv1, srcv2)
        dstv = (dstv0, dstv1, dstv2)
        rows = (rows0, rows1, rows2)
        semi = (semi0, semi1, semi2)
        semg = (semg0, semg1, semg2)
        zeros16 = jnp.zeros((16,), jnp.float32)

        def _idx_copy(jj, b):
            return (pltpu.make_async_copy(src4.at[c, s, jj], srcv[b], semi[b]),
                    pltpu.make_async_copy(dst4.at[c, s, jj], dstv[b], semi[b]))

        def _gather(b):
            return pltpu.make_async_copy(
                hstack.at[srcv[b]], rows[b], semg[b])

        def _zero_row(r, carry):
            for k0 in range(8):
                rows0[r, pl.ds(k0 * 16, 16)] = zeros16
            return carry

        lax.fori_loop(0, CHUNK, _zero_row, 0)

        base = s * ROWS_PER_SUB
        pieces = []
        off = 0
        while off < ROWS_PER_SUB:
            sz = min(CHUNK, ROWS_PER_SUB - off)
            pieces.append((off, sz))
            off += sz
        for off, sz in pieces:
            pltpu.sync_copy(rows0.at[pl.ds(0, sz)], acc.at[pl.ds(base + off, sz)])
        plsc.subcore_barrier()

        for bb in range(3):
            for cp in _idx_copy(bb, bb):
                cp.start()
        for cp in _idx_copy(0, 0):
            cp.wait()
        _gather(0).start()
        for cp in _idx_copy(1, 1):
            cp.wait()
        _gather(1).start()

        def _edge_trip(i, carry):
            for b in range(3):
                jj = 3 * i + b
                nb2 = (b + 2) % 3

                @pl.when(jj + 2 < nch)
                def _next_gather():
                    for cp in _idx_copy(jj + 2, nb2):
                        cp.wait()
                    _gather(nb2).start()

                _gather(b).wait()
                pltpu.sync_copy(rows[b], acc.at[dstv[b]], add=True)

                @pl.when(jj + 3 < nch)
                def _next_idx():
                    for cp in _idx_copy(jj + 3, b):
                        cp.start()
            return carry

        lax.fori_loop(0, nch // 3, _edge_trip, 0)
        plsc.subcore_barrier()

        obase = c * N_ACC + base
        for off, sz in pieces:
            pltpu.sync_copy(acc.at[pl.ds(base + off, sz)], rows0.at[pl.ds(0, sz)])
            pltpu.sync_copy(rows0.at[pl.ds(0, sz)], out.at[pl.ds(obase + off, sz)])

    return segsum


def _edge_prep(edge_index):
    src = edge_index[0]
    dst = edge_index[1]
    pad_iota = jnp.arange(E_PAD - E, dtype=jnp.int32)
    srcp = jnp.concatenate([src, pad_iota % N])
    dstp = jnp.concatenate([dst, N + pad_iota % (N_ACC - N)])
    src_r = srcp.reshape(NS, NCH_C, CHUNK)
    dst_r = dstp.reshape(NS, NCH_C, CHUNK)
    sd_c = (jnp.stack([src_r, src_r + N]), jnp.stack([dst_r, dst_r]))
    sd_e = (srcp.reshape(NC, NS, NCH_E, CHUNK),
            dstp.reshape(NC, NS, NCH_E, CHUNK))
    return sd_c, sd_e


def _segsum_parts(h_parts, sd_c, sd_e):
    if len(h_parts) == 1:
        out = _make_sc_segsum(N, NCH_E)(*sd_e, h_parts[0])
        return [out[:N], out[N_ACC:N_ACC + N]], "sum"
    hstack = jnp.concatenate(h_parts, axis=0)
    out = _make_sc_segsum(2 * N, NCH_C)(*sd_c, hstack)
    return [out[:N], out[N_ACC:N_ACC + N]], "concat"


def kernel(x, edge_index, Wg0, bg0, eps0, Wg1, bg1, eps1, Wg2, bg2, eps2,
           Wd1, bd1, Wd2, bd2):
    edges_c, edges_e = _edge_prep(edge_index)
    readouts = []
    h_parts = [x]
    for (W, b, eps) in ((Wg0, bg0, eps0), (Wg1, bg1, eps1), (Wg2, bg2, eps2)):
        agg_parts, agg_mode = _segsum_parts(h_parts, edges_c, edges_e)
        e1 = jnp.reshape(1.0 + eps, (1, 1))
        y0, y1, s, m = _tc_layer(h_parts, agg_parts, W, b[None, :], e1,
                                 agg_mode=agg_mode)
        readouts.extend([s, m])
        h_parts = [y0, y1]
    merged = jnp.concatenate(readouts, axis=1)
    return _head(merged, Wd1, bd1, Wd2, bd2)
